# baseline (speedup 1.0000x reference)
import functools

import jax
import jax.numpy as jnp
import numpy as np
from jax import lax
from jax.experimental import pallas as pl
from jax.experimental.pallas import tpu as pltpu
from jax.experimental.pallas import tpu_sc as plsc

_N_BINS = 15
_STEP = np.float32(1.0 / _N_BINS)
_L = 16
_GROUPS = 25
_CHUNK = _GROUPS * _L
_NW = 32
_HIST = _N_BINS * _L + _L


def _sc_body(n, c, p_hbm, lab_hbm, out_hbm, buf, lbuf, cnt_h, scf_h, sac_h,
             sem, lsem):
    wid = lax.axis_index("s") * 2 + lax.axis_index("c")
    n_chunks = n // _CHUNK
    lo = wid * n_chunks // _NW
    hi = (wid + 1) * n_chunks // _NW

    lane = lax.iota(jnp.int32, _L)
    zero_v = jnp.zeros((_L,), jnp.float32)
    for r in (cnt_h, scf_h, sac_h):
        for b in range(_HIST // _L):
            r[pl.ds(b * _L, _L)] = zero_v

    ones_v = jnp.ones((_L,), jnp.float32)

    def chunk_body(ci, _):
        base = ci * _CHUNK
        cp = pltpu.make_async_copy(
            p_hbm.at[pl.ds(base * c, _CHUNK * c)], buf, sem)
        cp.start()
        cl = pltpu.make_async_copy(
            lab_hbm.at[pl.ds(base, _CHUNK)], lbuf, lsem)
        cl.start()
        cp.wait()
        cl.wait()

        def group_body(g):
            gbase = g * (_L * c) + lane * c
            conf = plsc.load_gather(buf, [gbase])
            for cc in range(1, c):
                v = plsc.load_gather(buf, [gbase + cc])
                conf = jnp.maximum(conf, v)
            lab_v = lbuf[pl.ds(g * _L, _L)]
            plab = plsc.load_gather(buf, [gbase + lab_v])
            accv = jnp.where(plab == conf, 1.0, 0.0).astype(jnp.float32)

            nb = jnp.zeros((_L,), jnp.int32)
            for k in range(_N_BINS):
                nb = nb + jnp.where(conf > np.float32(k) * _STEP, 1, 0)
            valid = nb >= 1
            b_idx = jnp.maximum(nb - 1, 0)
            off = b_idx * _L + lane
            plsc.addupdate_scatter(cnt_h, [off], ones_v, mask=valid)
            plsc.addupdate_scatter(scf_h, [off], conf, mask=valid)
            plsc.addupdate_scatter(sac_h, [off], accv, mask=valid)

        for g in range(_GROUPS):
            group_body(g)
        return 0

    lax.fori_loop(lo, hi, chunk_body, 0)

    obase = wid * 3 * _HIST
    pltpu.sync_copy(cnt_h, out_hbm.at[pl.ds(obase, _HIST)])
    pltpu.sync_copy(scf_h, out_hbm.at[pl.ds(obase + _HIST, _HIST)])
    pltpu.sync_copy(sac_h, out_hbm.at[pl.ds(obase + 2 * _HIST, _HIST)])


def kernel(probabilities, labels):
    n, c = probabilities.shape
    flat = probabilities.reshape(n * c)
    labs = labels.astype(jnp.int32)

    mesh = plsc.VectorSubcoreMesh(core_axis_name="c", subcore_axis_name="s")
    sc = pl.kernel(
        functools.partial(_sc_body, n, c),
        mesh=mesh,
        out_type=jax.ShapeDtypeStruct((_NW * 3 * _HIST,), jnp.float32),
        scratch_types=[
            pltpu.VMEM((_CHUNK * c,), jnp.float32),
            pltpu.VMEM((_CHUNK,), jnp.int32),
            pltpu.VMEM((_HIST,), jnp.float32),
            pltpu.VMEM((_HIST,), jnp.float32),
            pltpu.VMEM((_HIST,), jnp.float32),
            pltpu.SemaphoreType.DMA,
            pltpu.SemaphoreType.DMA,
        ],
        compiler_params=pltpu.CompilerParams(needs_layout_passes=False),
    )
    hists = sc(flat, labs)

    h = hists.reshape(_NW, 3, _HIST).sum(axis=0)
    h = h.reshape(3, _HIST // _L, _L).sum(axis=2)
    cnt = h[0, :_N_BINS]
    sconf = h[1, :_N_BINS]
    sacc = h[2, :_N_BINS]
    nonempty = cnt > 0
    safe = jnp.where(nonempty, cnt, 1.0)
    per_bin = jnp.where(
        nonempty,
        jnp.abs(sconf / safe - sacc / safe) * (cnt * (1.0 / n)),
        0.0,
    )
    return jnp.sum(per_bin).reshape(1)


# speedup vs baseline: 1.0758x; 1.0758x over previous
import functools

import jax
import jax.numpy as jnp
import numpy as np
from jax import lax
from jax.experimental import pallas as pl
from jax.experimental.pallas import tpu as pltpu
from jax.experimental.pallas import tpu_sc as plsc

_N_BINS = 15
_STEP = np.float32(1.0 / _N_BINS)
_L = 16
_GROUPS = 25
_CHUNK = _GROUPS * _L
_NW = 32
_HIST = _N_BINS * _L + _L


def _sc_body(n, c, p_hbm, lab_hbm, out_hbm, buf, lbuf, cnt_h, scf_h, sac_h,
             sem, lsem):
    wid = lax.axis_index("s") * 2 + lax.axis_index("c")
    n_chunks = n // _CHUNK
    lo = wid * n_chunks // _NW
    hi = (wid + 1) * n_chunks // _NW

    lane = lax.iota(jnp.int32, _L)
    zero_v = jnp.zeros((_L,), jnp.float32)
    for r in (cnt_h, scf_h, sac_h):
        for b in range(_HIST // _L):
            r[pl.ds(b * _L, _L)] = zero_v

    ones_v = jnp.ones((_L,), jnp.float32)

    def chunk_body(ci, _):
        base = ci * _CHUNK
        cp = pltpu.make_async_copy(
            p_hbm.at[pl.ds(base * c, _CHUNK * c)], buf, sem)
        cp.start()
        cl = pltpu.make_async_copy(
            lab_hbm.at[pl.ds(base, _CHUNK)], lbuf, lsem)
        cl.start()
        cp.wait()
        cl.wait()

        def group_body(g, _):
            gbase = g * (_L * c) + lane * c
            conf = plsc.load_gather(buf, [gbase])
            for cc in range(1, c):
                v = plsc.load_gather(buf, [gbase + cc])
                conf = jnp.maximum(conf, v)
            lab_v = lbuf[pl.ds(g * _L, _L)]
            plab = plsc.load_gather(buf, [gbase + lab_v])
            accv = jnp.where(plab == conf, 1.0, 0.0).astype(jnp.float32)

            nb = jnp.zeros((_L,), jnp.int32)
            for k in range(_N_BINS):
                nb = nb + jnp.where(conf > np.float32(k) * _STEP, 1, 0)
            valid = nb >= 1
            b_idx = jnp.maximum(nb - 1, 0)
            off = b_idx * _L + lane
            plsc.addupdate_scatter(cnt_h, [off], ones_v, mask=valid)
            plsc.addupdate_scatter(scf_h, [off], conf, mask=valid)
            plsc.addupdate_scatter(sac_h, [off], accv, mask=valid)
            return 0

        lax.fori_loop(0, _GROUPS, group_body, 0)
        return 0

    lax.fori_loop(lo, hi, chunk_body, 0)

    obase = wid * 3 * _HIST
    pltpu.sync_copy(cnt_h, out_hbm.at[pl.ds(obase, _HIST)])
    pltpu.sync_copy(scf_h, out_hbm.at[pl.ds(obase + _HIST, _HIST)])
    pltpu.sync_copy(sac_h, out_hbm.at[pl.ds(obase + 2 * _HIST, _HIST)])


def kernel(probabilities, labels):
    n, c = probabilities.shape
    flat = probabilities.reshape(n * c)
    labs = labels.astype(jnp.int32)

    mesh = plsc.VectorSubcoreMesh(core_axis_name="c", subcore_axis_name="s")
    sc = pl.kernel(
        functools.partial(_sc_body, n, c),
        mesh=mesh,
        out_type=jax.ShapeDtypeStruct((_NW * 3 * _HIST,), jnp.float32),
        scratch_types=[
            pltpu.VMEM((_CHUNK * c,), jnp.float32),
            pltpu.VMEM((_CHUNK,), jnp.int32),
            pltpu.VMEM((_HIST,), jnp.float32),
            pltpu.VMEM((_HIST,), jnp.float32),
            pltpu.VMEM((_HIST,), jnp.float32),
            pltpu.SemaphoreType.DMA,
            pltpu.SemaphoreType.DMA,
        ],
        compiler_params=pltpu.CompilerParams(needs_layout_passes=False),
    )
    hists = sc(flat, labs)

    h = hists.reshape(_NW, 3, _HIST).sum(axis=0)
    h = h.reshape(3, _HIST // _L, _L).sum(axis=2)
    cnt = h[0, :_N_BINS]
    sconf = h[1, :_N_BINS]
    sacc = h[2, :_N_BINS]
    nonempty = cnt > 0
    safe = jnp.where(nonempty, cnt, 1.0)
    per_bin = jnp.where(
        nonempty,
        jnp.abs(sconf / safe - sacc / safe) * (cnt * (1.0 / n)),
        0.0,
    )
    return jnp.sum(per_bin).reshape(1)
